# Initial kernel scaffold; baseline (speedup 1.0000x reference)
import jax
import jax.numpy as jnp
from jax.experimental import pallas as pl


def kernel(pts, params):
    raise NotImplementedError("write your pallas kernel here")


# speedup vs baseline: 9.4959x; 9.4959x over previous
import functools

import jax
import jax.numpy as jnp
from jax import lax
from jax.experimental import pallas as pl
from jax.experimental.pallas import tpu as pltpu
from jax.experimental.pallas import tpu_sc as plsc

KNB = 16
CS = 128
RB = 256
RD = 256

_NC = 2
_NS = 16
_NW = _NC * _NS
_CH = 128
_GRP = 4


def _silu(x):
    return x * jax.nn.sigmoid(x)


def _knn_body(state_ref, idx_ref, *, n):
    b = pl.program_id(0)
    i = pl.program_id(1)
    x_all = state_ref[0, :, 64:70]
    x_row = state_ref[0, pl.ds(i * RB, RB), 64:70]
    sq_c = jnp.sum(x_all * x_all, axis=1)[None, :]
    sq_r = jnp.sum(x_row * x_row, axis=1, keepdims=True)
    inner = lax.dot_general(x_row, x_all, (((1,), (1,)), ((), ())),
                            preferred_element_type=jnp.float32)
    d2 = (sq_r + sq_c) - 2.0 * inner
    vals = jnp.sqrt(jnp.maximum(d2, 0.0))

    col = lax.broadcasted_iota(jnp.int32, (RB, n), 1)
    lane = lax.broadcasted_iota(jnp.int32, (RB, KNB), 1)
    acc = jnp.zeros((RB, KNB), jnp.int32)
    neg = jnp.float32(-jnp.inf)
    for t in range(KNB + 1):
        m = jnp.max(vals, axis=1, keepdims=True)
        am = jnp.min(jnp.where(vals == m, col, n), axis=1, keepdims=True)
        if t > 0:
            acc = jnp.where(lane == (t - 1), am, acc)
        if t < KNB:
            vals = jnp.where(col == am, neg, vals)
    idx_ref[0, :, :] = acc + b * n


def _knn_call(state):
    b, n, _ = state.shape
    body = functools.partial(_knn_body, n=n)
    return pl.pallas_call(
        body,
        grid=(b, n // RB),
        in_specs=[pl.BlockSpec((1, n, CS), lambda bi, ri: (bi, 0, 0))],
        out_specs=pl.BlockSpec((1, RB, KNB), lambda bi, ri: (bi, ri, 0)),
        out_shape=jax.ShapeDtypeStruct((b, n, KNB), jnp.int32),
    )(state)


def _gather_body(table_hbm, idx_hbm, out_hbm, idx_v, rows_v, sem, *, nch):
    wid = lax.axis_index("s") * _NC + lax.axis_index("c")
    pltpu.sync_copy(idx_hbm.at[wid], idx_v)
    for g in range(nch // _GRP):
        handles = []
        for j in range(_GRP):
            c = g * _GRP + j
            handles.append(
                pltpu.async_copy(table_hbm.at[idx_v.at[c]], rows_v.at[j], sem))
        for h in handles:
            h.wait()
        pltpu.sync_copy(rows_v, out_hbm.at[pl.ds(wid * nch + g * _GRP, _GRP)])


def _gather_sc(table, idx_flat):
    tot = idx_flat.shape[0]
    per_w = tot // _NW
    nch = per_w // _CH
    idx3 = idx_flat.reshape(_NW, nch, _CH)
    mesh = plsc.VectorSubcoreMesh(core_axis_name="c", subcore_axis_name="s")
    body = functools.partial(_gather_body, nch=nch)
    out3 = pl.kernel(
        body,
        mesh=mesh,
        out_type=jax.ShapeDtypeStruct((tot // _CH, _CH, CS), jnp.float32),
        scratch_types=[
            pltpu.VMEM((nch, _CH), jnp.int32),
            pltpu.VMEM((_GRP, _CH, CS), jnp.float32),
            pltpu.SemaphoreType.DMA,
        ],
    )(table, idx3)
    return out3.reshape(tot, CS)


def _dense_body(state_ref, gath_ref, we_ref, wf_ref, wx_ref,
                be_ref, bf_ref, bx_ref, out_ref, *, first):
    fs = state_ref[0, :, 0:64]
    xs = state_ref[0, :, 64:70]
    g = gath_ref[0]
    fn = g[:, 0:64]
    xn = g[:, 64:70]

    xd = xn.reshape(RD, KNB, 6) - xs.reshape(RD, 1, 6)
    xdsq = jnp.sum(xd * xd, axis=2, keepdims=True)

    fsb = jnp.broadcast_to(fs.reshape(RD, 1, 64),
                           (RD, KNB, 64)).reshape(RD * KNB, 64)
    feat = jnp.concatenate([
        fn - fsb, fsb, xdsq.reshape(RD * KNB, 1),
        jnp.zeros((RD * KNB, 7), jnp.float32)], axis=1)
    m = _silu(jnp.dot(feat, we_ref[...],
                      preferred_element_type=jnp.float32) + be_ref[0])
    summ = jnp.sum(m.reshape(RD, KNB, 64), axis=1)

    phi = _silu(jnp.dot(m, wx_ref[...],
                        preferred_element_type=jnp.float32) + bx_ref[0])
    phi3 = phi.reshape(RD, KNB, 2)
    u0 = jnp.mean(xd[:, :, 0:3] * phi3[:, :, 0:1], axis=1)
    xd1 = xd[:, :, 0:3] if first else xd[:, :, 3:6]
    u1 = jnp.mean(xd1 * phi3[:, :, 1:2], axis=1)
    if first:
        base = jnp.concatenate([xs[:, 0:3], xs[:, 0:3]], axis=1)
    else:
        base = xs
    xnew = base + jnp.concatenate([u0, u1], axis=1)

    fin = jnp.concatenate([fs, summ], axis=1)
    fnew = _silu(jnp.dot(fin, wf_ref[...],
                         preferred_element_type=jnp.float32) + bf_ref[0])
    out_ref[0] = jnp.concatenate(
        [fnew, xnew, jnp.zeros((RD, CS - 70), jnp.float32)], axis=1)


def _dense_call(state, gath, lp, first):
    b, n, _ = state.shape
    wspec = lambda shp: pl.BlockSpec(shp, lambda bi, ri: (0,) * len(shp))
    body = functools.partial(_dense_body, first=first)
    return pl.pallas_call(
        body,
        grid=(b, n // RD),
        in_specs=[
            pl.BlockSpec((1, RD, CS), lambda bi, ri: (bi, ri, 0)),
            pl.BlockSpec((1, RD * KNB, CS), lambda bi, ri: (bi, ri, 0)),
            wspec((136, 64)), wspec((128, 64)), wspec((64, 2)),
            wspec((1, 64)), wspec((1, 64)), wspec((1, 2)),
        ],
        out_specs=pl.BlockSpec((1, RD, CS), lambda bi, ri: (bi, ri, 0)),
        out_shape=jax.ShapeDtypeStruct((b, n, CS), jnp.float32),
    )(state, gath.reshape(b, n * KNB, CS),
      lp['we'], lp['wf'], lp['wx'], lp['be'], lp['bf'], lp['bx'])


def _prep_layers(p):
    we1 = jnp.zeros((136, 64), jnp.float32)
    we1 = we1.at[64].set(p['e1_w'][:, 1]).at[128].set(p['e1_w'][:, 2])
    wf1 = jnp.zeros((128, 64), jnp.float32)
    wf1 = wf1.at[0].set(p['f1_w'][:, 0]).at[64:128].set(p['f1_w'][:, 1:65].T)
    layers = [dict(
        we=we1,
        be=p['e1_b'].reshape(1, 64),
        wx=p['x1_w'].T,
        bx=p['x1_b'].reshape(1, 2),
        wf=wf1,
        bf=p['f1_b'].reshape(1, 64),
    )]
    for i in (2, 3, 4):
        layers.append(dict(
            we=jnp.concatenate(
                [p[f'e{i}_w'].T, jnp.zeros((7, 64), jnp.float32)], axis=0),
            be=p[f'e{i}_b'].reshape(1, 64),
            wx=p[f'x{i}_w'].T,
            bx=p[f'x{i}_b'].reshape(1, 2),
            wf=p[f'f{i}_w'].T,
            bf=p[f'f{i}_b'].reshape(1, 64),
        ))
    return layers


def kernel(pts, params):
    b, _, n = pts.shape
    xt = jnp.transpose(pts, (0, 2, 1))
    ones = jnp.ones((b, n, 1), jnp.float32)
    state = jnp.concatenate([
        ones,
        jnp.zeros((b, n, 63), jnp.float32),
        xt,
        jnp.zeros((b, n, CS - 67), jnp.float32),
    ], axis=2)
    layers = _prep_layers(params)
    for li, lp in enumerate(layers):
        idx = _knn_call(state)
        gath = _gather_sc(state.reshape(b * n, CS),
                          idx.reshape(b * n * KNB))
        state = _dense_call(state, gath, lp, first=(li == 0))
    x_out = jnp.transpose(state[:, :, 64:70], (0, 2, 1))
    f_out = jnp.transpose(state[:, :, 0:64], (0, 2, 1))
    return x_out, f_out
